# Initial kernel scaffold; baseline (speedup 1.0000x reference)
import functools

import jax
import jax.numpy as jnp
from jax import lax
from jax.experimental import pallas as pl
from jax.experimental.pallas import tpu as pltpu
from jax.experimental.pallas import tpu_sc as plsc

B, L, D = 4096, 50, 32
N = B * L
NC, NS = 2, 16
NW = NC * NS
BPW = N // NW
CH = 1600
NCH = BPW // CH


def _make_emb_kernel(interpret=False):
    mesh = plsc.VectorSubcoreMesh(core_axis_name="c", subcore_axis_name="s",
                                  num_cores=NC, num_subcores=NS)

    @functools.partial(
        pl.kernel,
        out_type=(jax.ShapeDtypeStruct((N, D), jnp.float32),
                  jax.ShapeDtypeStruct((N, D), jnp.float32)),
        mesh=mesh,
        scratch_types=[
            pltpu.VMEM((CH,), jnp.int32),
            pltpu.VMEM((CH, D), jnp.float32),
            pltpu.SemaphoreType.DMA,
        ],
        interpret=interpret,
    )
    def emb_kernel(src_tok, tgt_tok, src_tab, tgt_tab, src_out, tgt_out,
                   idx_v, rows_v, sem):
        wid = lax.axis_index("s") * NC + lax.axis_index("c")
        base = wid * BPW
        for tok, tab, out in ((src_tok, src_tab, src_out),
                              (tgt_tok, tgt_tab, tgt_out)):
            for c in range(NCH):
                off = base + c * CH
                pltpu.sync_copy(tok.at[pl.ds(off, CH)], idx_v)
                pltpu.async_copy(tab.at[idx_v], rows_v, sem).wait()
                pltpu.sync_copy(rows_v, out.at[pl.ds(off, CH)])

    return emb_kernel


_emb = _make_emb_kernel()


def kernel(src_tokens, tgt_tokens, src_table, tgt_table):
    src_flat = src_tokens.reshape(N).astype(jnp.int32)
    tgt_flat = tgt_tokens.reshape(N).astype(jnp.int32)
    s, t = _emb(src_flat, tgt_flat, src_table, tgt_table)
    return s.reshape(B, L, D), t.reshape(B, L, D)


# speedup vs baseline: 3.0574x; 3.0574x over previous
import functools

import jax
import jax.numpy as jnp
from jax import lax
from jax.experimental import pallas as pl
from jax.experimental.pallas import tpu as pltpu
from jax.experimental.pallas import tpu_sc as plsc

B, L, D = 4096, 50, 32
N = B * L
NC, NS = 2, 16
NW = NC * NS
BPW = N // NW
CH = 1600
NCH = BPW // CH


def _make_emb_kernel(interpret=False):
    mesh = plsc.VectorSubcoreMesh(core_axis_name="c", subcore_axis_name="s",
                                  num_cores=NC, num_subcores=NS)

    @functools.partial(
        pl.kernel,
        out_type=(jax.ShapeDtypeStruct((N, D), jnp.float32),
                  jax.ShapeDtypeStruct((N, D), jnp.float32)),
        mesh=mesh,
        scratch_types=[
            pltpu.VMEM((CH,), jnp.int32),
            pltpu.VMEM((CH, D), jnp.float32),
            pltpu.SemaphoreType.DMA,
        ],
        compiler_params=pltpu.CompilerParams(use_tc_tiling_on_sc=False),
        interpret=interpret,
    )
    def emb_kernel(src_tok, tgt_tok, src_tab, tgt_tab, src_out, tgt_out,
                   idx_v, rows_v, sem):
        wid = lax.axis_index("s") * NC + lax.axis_index("c")
        base = wid * BPW
        for tok, tab, out in ((src_tok, src_tab, src_out),
                              (tgt_tok, tgt_tab, tgt_out)):
            for c in range(NCH):
                off = base + c * CH
                pltpu.sync_copy(tok.at[pl.ds(off, CH)], idx_v)
                pltpu.async_copy(tab.at[idx_v], rows_v, sem).wait()
                pltpu.sync_copy(rows_v, out.at[pl.ds(off, CH)])

    return emb_kernel


_emb = _make_emb_kernel()


def kernel(src_tokens, tgt_tokens, src_table, tgt_table):
    src_flat = src_tokens.reshape(N).astype(jnp.int32)
    tgt_flat = tgt_tokens.reshape(N).astype(jnp.int32)
    s, t = _emb(src_flat, tgt_flat, src_table, tgt_table)
    return s.reshape(B, L, D), t.reshape(B, L, D)


# speedup vs baseline: 3.0880x; 1.0100x over previous
import functools

import jax
import jax.numpy as jnp
from jax import lax
from jax.experimental import pallas as pl
from jax.experimental.pallas import tpu as pltpu
from jax.experimental.pallas import tpu_sc as plsc

B, L, D = 4096, 50, 32
N = B * L
NC, NS = 2, 16
NW = NC * NS
BPW = N // NW
CH = 1600
NCH = BPW // CH
NB = 2


def _make_emb_kernel(interpret=False):
    mesh = plsc.VectorSubcoreMesh(core_axis_name="c", subcore_axis_name="s",
                                  num_cores=NC, num_subcores=NS)

    @functools.partial(
        pl.kernel,
        out_type=(jax.ShapeDtypeStruct((N, D), jnp.float32),
                  jax.ShapeDtypeStruct((N, D), jnp.float32)),
        mesh=mesh,
        scratch_types=[
            pltpu.VMEM((2, BPW), jnp.int32),
            pltpu.VMEM((NB, CH, D), jnp.float32),
            pltpu.SemaphoreType.DMA((NB,)),
            pltpu.SemaphoreType.DMA((NB,)),
            pltpu.SemaphoreType.DMA,
        ],
        compiler_params=pltpu.CompilerParams(use_tc_tiling_on_sc=False),
        interpret=interpret,
    )
    def emb_kernel(src_tok, tgt_tok, src_tab, tgt_tab, src_out, tgt_out,
                   idx_v, rows_v, gsem, wsem, isem):
        wid = lax.axis_index("s") * NC + lax.axis_index("c")
        base = wid * BPW
        i0 = pltpu.async_copy(src_tok.at[pl.ds(base, BPW)], idx_v.at[0], isem)
        i1 = pltpu.async_copy(tgt_tok.at[pl.ds(base, BPW)], idx_v.at[1], isem)
        i0.wait()
        i1.wait()

        def gather(t, c, b, tab):
            return pltpu.async_copy(
                tab.at[idx_v.at[t, pl.ds(c * CH, CH)]], rows_v.at[b],
                gsem.at[b])

        def writeback(c, b, out):
            return pltpu.async_copy(
                rows_v.at[b], out.at[pl.ds(base + c * CH, CH)], wsem.at[b])

        for t, (tab, out) in enumerate(((src_tab, src_out),
                                        (tgt_tab, tgt_out))):
            g = [gather(t, c, c % NB, tab) for c in range(min(NB, NCH))]
            w = [None] * NCH
            for c in range(NCH):
                b = c % NB
                g[b].wait()
                w[c] = writeback(c, b, out)
                if c + NB < NCH:
                    w[c].wait()
                    g[b] = gather(t, c + NB, b, tab)
            for c in range(max(0, NCH - NB), NCH):
                if c + NB >= NCH:
                    w[c].wait()

    return emb_kernel


_emb = _make_emb_kernel()


def kernel(src_tokens, tgt_tokens, src_table, tgt_table):
    src_flat = src_tokens.reshape(N).astype(jnp.int32)
    tgt_flat = tgt_tokens.reshape(N).astype(jnp.int32)
    s, t = _emb(src_flat, tgt_flat, src_table, tgt_table)
    return s.reshape(B, L, D), t.reshape(B, L, D)


# speedup vs baseline: 5.1611x; 1.6713x over previous
import functools

import jax
import jax.numpy as jnp
from jax import lax
from jax.experimental import pallas as pl
from jax.experimental.pallas import tpu as pltpu
from jax.experimental.pallas import tpu_sc as plsc

B, L, D = 4096, 50, 32
NC, NS = 2, 16
NW = NC * NS
SPW = B // NW
NB = 8


def _make_emb_kernel(interpret=False):
    mesh = plsc.VectorSubcoreMesh(core_axis_name="c", subcore_axis_name="s",
                                  num_cores=NC, num_subcores=NS)

    @functools.partial(
        pl.kernel,
        out_type=(jax.ShapeDtypeStruct((B, L, D), jnp.float32),
                  jax.ShapeDtypeStruct((B, L, D), jnp.float32)),
        mesh=mesh,
        scratch_types=[
            pltpu.VMEM((2, SPW, L), jnp.int32),
            pltpu.VMEM((NB, L, D), jnp.float32),
            pltpu.SemaphoreType.DMA((NB,)),
            pltpu.SemaphoreType.DMA((NB,)),
        ],
        compiler_params=pltpu.CompilerParams(use_tc_tiling_on_sc=False),
        interpret=interpret,
    )
    def emb_kernel(src_tok, tgt_tok, src_tab, tgt_tab, src_out, tgt_out,
                   idx_v, ring, gsem, wsem):
        cc = lax.axis_index("c")
        ss = lax.axis_index("s")
        wid = ss * NC + cc
        base = wid * SPW

        pltpu.sync_copy(src_tok.at[pl.ds(base, SPW)], idx_v.at[0])
        pltpu.sync_copy(tgt_tok.at[pl.ds(base, SPW)], idx_v.at[1])

        for t, (tab, out) in enumerate(((src_tab, src_out),
                                        (tgt_tab, tgt_out))):
            def g_start(k, b, tab=tab, t=t):
                pltpu.async_copy(tab.at[idx_v.at[t, k]], ring.at[b],
                                 gsem.at[b])

            def g_wait(b, tab=tab):
                pltpu.make_async_copy(tab.at[pl.ds(0, L)], ring.at[b],
                                      gsem.at[b]).wait()

            def w_start(k, b, out=out):
                pltpu.async_copy(ring.at[b], out.at[base + k], wsem.at[b])

            def w_wait(k, b, out=out):
                pltpu.make_async_copy(ring.at[b], out.at[base + k],
                                      wsem.at[b]).wait()

            for b in range(NB):
                g_start(b, b)

            @pl.loop(0, SPW - NB, step=NB)
            def _(k0):
                for b in range(NB):
                    k = k0 + b
                    g_wait(b)
                    w_start(k, b)
                    w_wait(k, b)
                    g_start(k + NB, b)

            for b in range(NB):
                g_wait(b)
                w_start(SPW - NB + b, b)
            for b in range(NB):
                w_wait(SPW - NB + b, b)

    return emb_kernel


_emb = _make_emb_kernel()


def kernel(src_tokens, tgt_tokens, src_table, tgt_table):
    return _emb(src_tokens.astype(jnp.int32), tgt_tokens.astype(jnp.int32),
                src_table, tgt_table)


# speedup vs baseline: 5.4699x; 1.0598x over previous
import functools

import jax
import jax.numpy as jnp
from jax import lax
from jax.experimental import pallas as pl
from jax.experimental.pallas import tpu as pltpu
from jax.experimental.pallas import tpu_sc as plsc

B, L, D = 4096, 50, 32
NC, NS = 2, 16
NW = NC * NS
SPW = B // NW
NB = 8


def _make_emb_kernel(interpret=False):
    mesh = plsc.VectorSubcoreMesh(core_axis_name="c", subcore_axis_name="s",
                                  num_cores=NC, num_subcores=NS)

    @functools.partial(
        pl.kernel,
        out_type=jax.ShapeDtypeStruct((B, L, D), jnp.float32),
        mesh=mesh,
        scratch_types=[
            pltpu.VMEM((SPW, L), jnp.int32),
            pltpu.VMEM((NB, L, D), jnp.float32),
            pltpu.SemaphoreType.DMA((NB,)),
            pltpu.SemaphoreType.DMA((NB,)),
        ],
        compiler_params=pltpu.CompilerParams(use_tc_tiling_on_sc=False),
        interpret=interpret,
    )
    def emb_kernel(tok, tab, out, idx_v, ring, gsem, wsem):
        cc = lax.axis_index("c")
        ss = lax.axis_index("s")
        wid = ss * NC + cc
        base = wid * SPW

        pltpu.sync_copy(tok.at[pl.ds(base, SPW)], idx_v)

        def g_start(k, b):
            pltpu.async_copy(tab.at[idx_v.at[k]], ring.at[b], gsem.at[b])

        def g_wait(b):
            pltpu.make_async_copy(tab.at[pl.ds(0, L)], ring.at[b],
                                  gsem.at[b]).wait()

        def w_start(k, b):
            pltpu.async_copy(ring.at[b], out.at[base + k], wsem.at[b])

        def w_wait(k, b):
            pltpu.make_async_copy(ring.at[b], out.at[base + k],
                                  wsem.at[b]).wait()

        for b in range(NB):
            g_start(b, b)

        @pl.loop(0, SPW - NB, step=NB)
        def _(k0):
            for b in range(NB):
                k = k0 + b
                g_wait(b)
                w_start(k, b)
                w_wait(k, b)
                g_start(k + NB, b)

        for b in range(NB):
            g_wait(b)
            w_start(SPW - NB + b, b)
        for b in range(NB):
            w_wait(SPW - NB + b, b)

    return emb_kernel


_emb = _make_emb_kernel()


def kernel(src_tokens, tgt_tokens, src_table, tgt_table):
    src_emb = _emb(src_tokens.astype(jnp.int32), src_table)
    tgt_emb = _emb(tgt_tokens.astype(jnp.int32), tgt_table)
    return src_emb, tgt_emb
